# baseline (speedup 1.0000x reference)
import functools

import jax
import jax.numpy as jnp
from jax import lax
from jax.experimental import pallas as pl
from jax.experimental.pallas import tpu as pltpu
from jax.experimental.pallas import tpu_sc as plsc

M = 16384
K = 256
N = 8192
BN = 2736
NPAD = 3 * BN
BM = 512
NM = M // BM
NN = 3


def _rn_bf16(v):
    return v.astype(jnp.bfloat16).astype(jnp.float32)


def _argmin_body(x_ref, e_ref, idx_ref, loss_ref, minv, mini, minx):
    i = pl.program_id(0)
    j = pl.program_id(1)

    x = x_ref[...]
    e = e_ref[...]
    s = lax.dot_general(x, e, (((1,), (1,)), ((), ())),
                        preferred_element_type=jnp.float32)
    a2 = jnp.sum(x * x, axis=1, keepdims=True)
    b2 = jnp.sum(e * e, axis=1)[None, :]
    d2 = a2 + b2 - 2.0 * s
    col = jax.lax.broadcasted_iota(jnp.int32, (BM, BN), 1) + j * BN
    d2 = jnp.where(col < N, d2, jnp.inf)
    bmin = jnp.min(d2, axis=1)
    bidx = jnp.argmin(d2, axis=1).astype(jnp.int32) + j * BN

    @pl.when(j == 0)
    def _():
        mini[...] = bidx
        minv[...] = _rn_bf16(bmin)
        minx[...] = bmin

    @pl.when(j > 0)
    def _():
        better = bmin < minv[...]
        mini[...] = jnp.where(better, bidx, mini[...])
        minv[...] = _rn_bf16(jnp.where(better, bmin, minv[...]))
        minx[...] = jnp.where(better, bmin, minx[...])

    @pl.when(j == NN - 1)
    def _():
        idx_ref[...] = mini[...]
        part = jnp.sum(minx[...])

        @pl.when(i == 0)
        def _():
            loss_ref[0, 0] = part

        @pl.when(i > 0)
        def _():
            loss_ref[0, 0] = loss_ref[0, 0] + part


def _tc_argmin(flat, embed_pad):
    return pl.pallas_call(
        _argmin_body,
        grid=(NM, NN),
        in_specs=[
            pl.BlockSpec((BM, K), lambda i, j: (i, 0)),
            pl.BlockSpec((BN, K), lambda i, j: (j, 0)),
        ],
        out_specs=[
            pl.BlockSpec((BM,), lambda i, j: (i,)),
            pl.BlockSpec(memory_space=pltpu.SMEM),
        ],
        out_shape=[
            jax.ShapeDtypeStruct((M,), jnp.int32),
            jax.ShapeDtypeStruct((1, 1), jnp.float32),
        ],
        scratch_shapes=[
            pltpu.VMEM((BM,), jnp.float32),
            pltpu.VMEM((BM,), jnp.int32),
            pltpu.VMEM((BM,), jnp.float32),
        ],
        compiler_params=pltpu.CompilerParams(
            dimension_semantics=("arbitrary", "arbitrary")),
    )(flat, embed_pad)


_NW = 32
_ROWS_PER_W = M // _NW
_CHUNK = 128
_NCHUNK = _ROWS_PER_W // _CHUNK


def _gather_body(table_hbm, idx_hbm, out_hbm, idx_v, rows_v, sem):
    wid = lax.axis_index("s") * 2 + lax.axis_index("c")
    base = wid * _ROWS_PER_W
    for c in range(_NCHUNK):
        off = base + c * _CHUNK
        pltpu.sync_copy(idx_hbm.at[pl.ds(off, _CHUNK)], idx_v)
        pltpu.async_copy(table_hbm.at[idx_v], rows_v, sem).wait()
        pltpu.sync_copy(rows_v, out_hbm.at[pl.ds(off, _CHUNK)])


def _sc_gather(embed, idx):
    mesh = plsc.VectorSubcoreMesh(core_axis_name="c", subcore_axis_name="s")
    k = functools.partial(
        pl.kernel,
        mesh=mesh,
        out_type=jax.ShapeDtypeStruct((M, K), jnp.float32),
        scratch_types=[
            pltpu.VMEM((_CHUNK,), jnp.int32),
            pltpu.VMEM((_CHUNK, K), jnp.float32),
            pltpu.SemaphoreType.DMA,
        ],
    )(_gather_body)
    return k(embed, idx)


def kernel(inputs, embed):
    x = inputs.astype(jnp.float32)
    flat = jnp.transpose(x, (0, 2, 3, 4, 1)).reshape(M, K)
    embed_pad = jnp.pad(embed, ((0, NPAD - N), (0, 0)))
    idx, loss_sum = _tc_argmin(flat, embed_pad)
    rows = _sc_gather(embed, idx)
    quantized = rows.reshape(4, 4, 32, 32, K).transpose(0, 4, 1, 2, 3)
    loss = (0.25 / (M * K)) * loss_sum[0, 0]
    return loss, quantized, idx.reshape(4, 4, 32, 32)


# speedup vs baseline: 1.1006x; 1.1006x over previous
import functools

import jax
import jax.numpy as jnp
from jax import lax
from jax.experimental import pallas as pl
from jax.experimental.pallas import tpu as pltpu
from jax.experimental.pallas import tpu_sc as plsc

M = 16384
K = 256
N = 8192
BN = 2736
NPAD = 3 * BN
BM = 1024
NM = M // BM
NN = 3


def _rn_bf16(v):
    return v.astype(jnp.bfloat16).astype(jnp.float32)


def _argmin_body(x_ref, e_ref, idx_ref, loss_ref, minv, mini, minx):
    i = pl.program_id(0)
    j = pl.program_id(1)

    x = x_ref[...]
    e = e_ref[...]
    s = lax.dot_general(x, e, (((1,), (1,)), ((), ())),
                        preferred_element_type=jnp.float32)
    a2 = jnp.sum(x * x, axis=1, keepdims=True)
    b2 = jnp.sum(e * e, axis=1)
    col = jax.lax.broadcasted_iota(jnp.int32, (1, BN), 1) + j * BN
    b2 = jnp.where(col < N, b2[None, :], jnp.inf)
    d2 = a2 + b2 - 2.0 * s
    bmin = jnp.min(d2, axis=1)
    bidx = jnp.argmin(d2, axis=1).astype(jnp.int32) + j * BN

    @pl.when(j == 0)
    def _():
        mini[...] = bidx
        minv[...] = _rn_bf16(bmin)
        minx[...] = bmin

    @pl.when(j > 0)
    def _():
        better = bmin < minv[...]
        mini[...] = jnp.where(better, bidx, mini[...])
        minv[...] = _rn_bf16(jnp.where(better, bmin, minv[...]))
        minx[...] = jnp.where(better, bmin, minx[...])

    @pl.when(j == NN - 1)
    def _():
        idx_ref[...] = mini[...]
        part = jnp.sum(minx[...])

        @pl.when(i == 0)
        def _():
            loss_ref[0, 0] = part

        @pl.when(i > 0)
        def _():
            loss_ref[0, 0] = loss_ref[0, 0] + part


def _tc_argmin(flat, embed_pad):
    return pl.pallas_call(
        _argmin_body,
        grid=(NM, NN),
        in_specs=[
            pl.BlockSpec((BM, K), lambda i, j: (i, 0)),
            pl.BlockSpec((BN, K), lambda i, j: (j, 0)),
        ],
        out_specs=[
            pl.BlockSpec((BM,), lambda i, j: (i,)),
            pl.BlockSpec(memory_space=pltpu.SMEM),
        ],
        out_shape=[
            jax.ShapeDtypeStruct((M,), jnp.int32),
            jax.ShapeDtypeStruct((1, 1), jnp.float32),
        ],
        scratch_shapes=[
            pltpu.VMEM((BM,), jnp.float32),
            pltpu.VMEM((BM,), jnp.int32),
            pltpu.VMEM((BM,), jnp.float32),
        ],
        compiler_params=pltpu.CompilerParams(
            dimension_semantics=("arbitrary", "arbitrary")),
    )(flat, embed_pad)


_NW = 32
_ROWS_PER_W = M // _NW
_CHUNK = 128
_NCHUNK = _ROWS_PER_W // _CHUNK


def _gather_body(table_hbm, idx_hbm, out_hbm, idx_v, rows_v, sem):
    wid = lax.axis_index("s") * 2 + lax.axis_index("c")
    base = wid * _ROWS_PER_W
    for c in range(_NCHUNK):
        off = base + c * _CHUNK
        pltpu.sync_copy(idx_hbm.at[pl.ds(off, _CHUNK)], idx_v)
        pltpu.async_copy(table_hbm.at[idx_v], rows_v, sem).wait()
        pltpu.sync_copy(rows_v, out_hbm.at[pl.ds(off, _CHUNK)])


def _sc_gather(embed, idx):
    mesh = plsc.VectorSubcoreMesh(core_axis_name="c", subcore_axis_name="s")
    k = functools.partial(
        pl.kernel,
        mesh=mesh,
        out_type=jax.ShapeDtypeStruct((M, K), jnp.float32),
        scratch_types=[
            pltpu.VMEM((_CHUNK,), jnp.int32),
            pltpu.VMEM((_CHUNK, K), jnp.float32),
            pltpu.SemaphoreType.DMA,
        ],
    )(_gather_body)
    return k(embed, idx)


def kernel(inputs, embed):
    x = inputs.astype(jnp.float32)
    flat = jnp.transpose(x, (0, 2, 3, 4, 1)).reshape(M, K)
    embed_pad = jnp.pad(embed, ((0, NPAD - N), (0, 0)))
    idx, loss_sum = _tc_argmin(flat, embed_pad)
    rows = _sc_gather(embed, idx)
    quantized = rows.reshape(4, 4, 32, 32, K).transpose(0, 4, 1, 2, 3)
    loss = (0.25 / (M * K)) * loss_sum[0, 0]
    return loss, quantized, idx.reshape(4, 4, 32, 32)


# speedup vs baseline: 1.1488x; 1.0438x over previous
import functools

import jax
import jax.numpy as jnp
from jax import lax
from jax.experimental import pallas as pl
from jax.experimental.pallas import tpu as pltpu
from jax.experimental.pallas import tpu_sc as plsc

M = 16384
K = 256
N = 8192
BN = 2736
NPAD = 3 * BN
BM = 2048
NM = M // BM
NN = 3


def _rn_bf16(v):
    return v.astype(jnp.bfloat16).astype(jnp.float32)


def _argmin_body(x_ref, e_ref, idx_ref, loss_ref, minv, mini, minx):
    i = pl.program_id(0)
    j = pl.program_id(1)

    x = x_ref[...]
    e = e_ref[...]
    s = lax.dot_general(x, e, (((1,), (1,)), ((), ())),
                        preferred_element_type=jnp.float32)
    a2 = jnp.sum(x * x, axis=1, keepdims=True)
    b2 = jnp.sum(e * e, axis=1)
    col = jax.lax.broadcasted_iota(jnp.int32, (1, BN), 1) + j * BN
    b2 = jnp.where(col < N, b2[None, :], jnp.inf)
    d2 = a2 + b2 - 2.0 * s
    bmin = jnp.min(d2, axis=1)
    bidx = jnp.argmin(d2, axis=1).astype(jnp.int32) + j * BN

    @pl.when(j == 0)
    def _():
        mini[...] = bidx
        minv[...] = _rn_bf16(bmin)
        minx[...] = bmin

    @pl.when(j > 0)
    def _():
        better = bmin < minv[...]
        mini[...] = jnp.where(better, bidx, mini[...])
        minv[...] = _rn_bf16(jnp.where(better, bmin, minv[...]))
        minx[...] = jnp.where(better, bmin, minx[...])

    @pl.when(j == NN - 1)
    def _():
        idx_ref[...] = mini[...]
        part = jnp.sum(minx[...])

        @pl.when(i == 0)
        def _():
            loss_ref[0, 0] = part

        @pl.when(i > 0)
        def _():
            loss_ref[0, 0] = loss_ref[0, 0] + part


def _tc_argmin(flat, embed_pad):
    return pl.pallas_call(
        _argmin_body,
        grid=(NM, NN),
        in_specs=[
            pl.BlockSpec((BM, K), lambda i, j: (i, 0)),
            pl.BlockSpec((BN, K), lambda i, j: (j, 0)),
        ],
        out_specs=[
            pl.BlockSpec((BM,), lambda i, j: (i,)),
            pl.BlockSpec(memory_space=pltpu.SMEM),
        ],
        out_shape=[
            jax.ShapeDtypeStruct((M,), jnp.int32),
            jax.ShapeDtypeStruct((1, 1), jnp.float32),
        ],
        scratch_shapes=[
            pltpu.VMEM((BM,), jnp.float32),
            pltpu.VMEM((BM,), jnp.int32),
            pltpu.VMEM((BM,), jnp.float32),
        ],
        compiler_params=pltpu.CompilerParams(
            dimension_semantics=("arbitrary", "arbitrary")),
    )(flat, embed_pad)


_NW = 32
_ROWS_PER_W = M // _NW
_CHUNK = 128
_NCHUNK = _ROWS_PER_W // _CHUNK


def _gather_body(table_hbm, idx_hbm, out_hbm, idx_v, rows_v, sem):
    wid = lax.axis_index("s") * 2 + lax.axis_index("c")
    base = wid * _ROWS_PER_W
    for c in range(_NCHUNK):
        off = base + c * _CHUNK
        pltpu.sync_copy(idx_hbm.at[pl.ds(off, _CHUNK)], idx_v)
        pltpu.async_copy(table_hbm.at[idx_v], rows_v, sem).wait()
        pltpu.sync_copy(rows_v, out_hbm.at[pl.ds(off, _CHUNK)])


def _sc_gather(embed, idx):
    mesh = plsc.VectorSubcoreMesh(core_axis_name="c", subcore_axis_name="s")
    k = functools.partial(
        pl.kernel,
        mesh=mesh,
        out_type=jax.ShapeDtypeStruct((M, K), jnp.float32),
        scratch_types=[
            pltpu.VMEM((_CHUNK,), jnp.int32),
            pltpu.VMEM((_CHUNK, K), jnp.float32),
            pltpu.SemaphoreType.DMA,
        ],
    )(_gather_body)
    return k(embed, idx)


def kernel(inputs, embed):
    x = inputs.astype(jnp.float32)
    flat = jnp.transpose(x, (0, 2, 3, 4, 1)).reshape(M, K)
    embed_pad = jnp.pad(embed, ((0, NPAD - N), (0, 0)))
    idx, loss_sum = _tc_argmin(flat, embed_pad)
    rows = _sc_gather(embed, idx)
    quantized = rows.reshape(4, 4, 32, 32, K).transpose(0, 4, 1, 2, 3)
    loss = (0.25 / (M * K)) * loss_sum[0, 0]
    return loss, quantized, idx.reshape(4, 4, 32, 32)
